# baseline (speedup 1.0000x reference)
import jax
import jax.numpy as jnp
from jax import lax
from jax.experimental import pallas as pl
from jax.experimental.pallas import tpu as pltpu
from jax.experimental.pallas import tpu_sc as plsc

_FRAME_TOPK = 16
_TOKEN_TOPK = 49
_EMA_ALPHA = 0.9
_NP = 208
_NEG = -3.0e38


def _argmax_col(v, iota_col, size):
    m = jnp.max(v, axis=0, keepdims=True)
    cand = jnp.where(v == m, iota_col, size)
    idx = jnp.min(cand, axis=0, keepdims=True)
    oh = (iota_col == idx).astype(jnp.float32)
    return oh, idx


def _dot(a, b, precision=jax.lax.Precision.HIGHEST):
    return jax.lax.dot_general(a, b, (((1,), (0,)), ((), ())),
                               preferred_element_type=jnp.float32,
                               precision=precision)


def _frame_repr_kernel(x_ref, out_ref):
    out_ref[...] = jnp.sum(x_ref[...], axis=2) / 196.0


def _frame_sel_kernel(fr_ref, idx_ref, mask_ref):
    T = 64
    fr = fr_ref[0]
    iota_col = jax.lax.broadcasted_iota(jnp.int32, (T, 1), 0)
    iota_row = jax.lax.broadcasted_iota(jnp.int32, (1, T), 1)
    iota_k = jax.lax.broadcasted_iota(jnp.int32, (1, _FRAME_TOPK), 1)

    def ema_body(t, carry):
        ema, nov = carry
        ft = fr_ref[0, pl.ds(t, 1), :]
        d = ft - ema
        n = jnp.sum(d * d)
        nov = jnp.where(iota_col == t, n, nov)
        ema = _EMA_ALPHA * ema + (1.0 - _EMA_ALPHA) * ft
        return ema, nov

    ema0 = fr_ref[0, 0:1, :]
    nov0 = jnp.zeros((T, 1), dtype=jnp.float32)
    _, nov = jax.lax.fori_loop(0, T, ema_body, (ema0, nov0))

    oh, idx = _argmax_col(nov, iota_col, T)
    idx_acc = jnp.where(iota_k == 0, idx, 0)
    mask_row = (iota_row == idx).astype(jnp.float32)
    min_d = jnp.full((T, 1), jnp.inf, dtype=jnp.float32)

    def fps_body(i, carry):
        oh, idx, idx_acc, mask_row, min_d = carry
        last = jnp.sum(oh * fr, axis=0, keepdims=True)
        diff = fr - last
        dist = jnp.sum(diff * diff, axis=1, keepdims=True)
        min_d = jnp.minimum(min_d, dist)
        min_d = jnp.where(oh > 0, -1.0, min_d)
        oh, idx = _argmax_col(min_d, iota_col, T)
        idx_acc = jnp.where(iota_k == i, idx, idx_acc)
        mask_row = mask_row + (iota_row == idx).astype(jnp.float32)
        return oh, idx, idx_acc, mask_row, min_d

    _, _, idx_acc, mask_row, _ = jax.lax.fori_loop(
        1, _FRAME_TOPK, fps_body, (oh, idx, idx_acc, mask_row, min_d))
    idx_ref[0] = idx_acc
    mask_ref[0] = mask_row


def _d2_kernel(fidx_ref, x_ref, d2_ref, d0_ref, f_ref):
    del fidx_ref
    F = x_ref[0, 0]
    Fp = jnp.concatenate(
        [F, jnp.zeros((_NP - 196, F.shape[1]), jnp.float32)], axis=0)
    f_ref[0] = Fp
    g = jax.lax.dot_general(Fp, Fp, (((1,), (1,)), ((), ())),
                            preferred_element_type=jnp.float32,
                            precision=jax.lax.Precision.HIGHEST)
    ii = jax.lax.broadcasted_iota(jnp.int32, (_NP, _NP), 0)
    jj = jax.lax.broadcasted_iota(jnp.int32, (_NP, _NP), 1)
    eye = (ii == jj).astype(jnp.float32)
    rn_col = jnp.sum(g * eye, axis=1, keepdims=True)
    rn_row = jnp.sum(g * eye, axis=0, keepdims=True)
    d2 = rn_col + rn_row - 2.0 * g
    pad = jnp.logical_or(ii >= 196, jj >= 196)
    d2_ref[0] = jnp.where(pad, _NEG, d2)

    gm_row = jnp.sum(g, axis=0, keepdims=True) * (1.0 / 196.0)
    musq = jnp.sum(g) * (1.0 / (196.0 * 196.0))
    d0 = rn_row - 2.0 * gm_row + musq
    jr = jax.lax.broadcasted_iota(jnp.int32, (1, _NP), 1)
    d0_ref[0] = jnp.where(jr >= 196, _NEG, d0)


def _mask_place_kernel(fidx_ref, cm_ref, out_ref):
    T = 64
    fidx = fidx_ref[0]
    cm = cm_ref[0]
    iota_col = jax.lax.broadcasted_iota(jnp.int32, (T, _FRAME_TOPK), 0)
    p = (iota_col == fidx).astype(jnp.float32)
    out_ref[0] = _dot(p, cm, precision=jax.lax.Precision.DEFAULT)


def _fps_tok_sc_body(d2_hbm, d0_hbm, f_hbm,
                     tidx_hbm, cm_hbm, z_hbm,
                     d2_v, d0_v, tidx_v, idx_v, cm_v, z_v, sem):
    K = _TOKEN_TOPK
    C = _NP // 16
    p = lax.axis_index("s") * 2 + lax.axis_index("c")
    pltpu.sync_copy(d2_hbm.at[p], d2_v)
    pltpu.sync_copy(d0_hbm.at[p], d0_v)
    iota = lax.iota(jnp.int32, 16)
    off = p * _NP

    def store_tidx(k, sel):
        for c in range(4):
            cur = tidx_v[pl.ds(c * 16, 16)]
            tidx_v[pl.ds(c * 16, 16)] = jnp.where(
                iota + (c * 16) == k, sel, cur)

    for c in range(4):
        tidx_v[pl.ds(c * 16, 16)] = jnp.zeros((16,), jnp.int32)

    def argmax_chunks(vecs):
        vmax = vecs[0]
        for v in vecs[1:]:
            vmax = jnp.maximum(vmax, v)
        gmax = vmax[0]
        for l in range(1, 16):
            gmax = jnp.maximum(gmax, vmax[l])
        cmin = jnp.where(vecs[0] == gmax, iota, _NP)
        for c in range(1, len(vecs)):
            cand = jnp.where(vecs[c] == gmax, iota + (c * 16), _NP)
            cmin = jnp.minimum(cmin, cand)
        sel = cmin[0]
        for l in range(1, 16):
            sel = jnp.minimum(sel, cmin[l])
        return sel

    sel0 = argmax_chunks([d0_v[pl.ds(c * 16, 16)] for c in range(C)])
    store_tidx(0, sel0)
    inf16 = jnp.full((16,), jnp.inf, jnp.float32)

    def body(k, carry):
        sel = carry[0]
        mins = carry[1:]
        new_mins = []
        for c in range(C):
            row = d2_v[sel, pl.ds(c * 16, 16)]
            m = jnp.minimum(mins[c], row)
            m = jnp.where(iota + (c * 16) == sel, -1.0, m)
            new_mins.append(m)
        new = argmax_chunks(new_mins)
        store_tidx(k, new)
        return (new, *new_mins)

    final = lax.fori_loop(1, K, body, (sel0, *([inf16] * C)))
    last_sel = final[0]
    for c in range(C):
        m = jnp.where(iota + (c * 16) == last_sel, -1.0, final[1 + c])
        cm_v[pl.ds(c * 16, 16)] = jnp.where(m == -1.0, 1.0, 0.0)

    for c in range(4):
        idx_v[pl.ds(c * 16, 16)] = tidx_v[pl.ds(c * 16, 16)] + off
    pltpu.async_copy(f_hbm.at[idx_v], z_v, sem).wait()

    pltpu.sync_copy(z_v, z_hbm.at[p])
    pltpu.sync_copy(cm_v, cm_hbm.at[p])
    pltpu.sync_copy(tidx_v, tidx_hbm.at[p])


def _run_fps_tok_sc(d2, d0, f_all):
    P = d2.shape[0]
    D = f_all.shape[1]
    mesh = plsc.VectorSubcoreMesh(core_axis_name="c", subcore_axis_name="s")
    fps = pl.kernel(
        _fps_tok_sc_body,
        out_type=[
            jax.ShapeDtypeStruct((P, 64), jnp.int32),
            jax.ShapeDtypeStruct((P, _NP), jnp.float32),
            jax.ShapeDtypeStruct((P, 64, D), jnp.float32),
        ],
        mesh=mesh,
        scratch_types=[
            pltpu.VMEM((_NP, _NP), jnp.float32),
            pltpu.VMEM((_NP,), jnp.float32),
            pltpu.VMEM((64,), jnp.int32),
            pltpu.VMEM((64,), jnp.int32),
            pltpu.VMEM((_NP,), jnp.float32),
            pltpu.VMEM((64, D), jnp.float32),
            pltpu.SemaphoreType.DMA,
        ],
    )
    return fps(d2, d0.reshape(P, _NP), f_all)


def kernel(x):
    B, T, N, D = x.shape
    K = _TOKEN_TOPK
    P = B * _FRAME_TOPK

    frame_repr = pl.pallas_call(
        _frame_repr_kernel,
        grid=(B, T // 8),
        in_specs=[pl.BlockSpec((1, 8, N, D), lambda b, t: (b, t, 0, 0))],
        out_specs=pl.BlockSpec((1, 8, D), lambda b, t: (b, t, 0)),
        out_shape=jax.ShapeDtypeStruct((B, T, D), jnp.float32),
    )(x)

    frame_idx3, frame_mask3 = pl.pallas_call(
        _frame_sel_kernel,
        grid=(B,),
        in_specs=[pl.BlockSpec((1, T, D), lambda b: (b, 0, 0))],
        out_specs=[
            pl.BlockSpec((1, 1, _FRAME_TOPK), lambda b: (b, 0, 0)),
            pl.BlockSpec((1, 1, T), lambda b: (b, 0, 0)),
        ],
        out_shape=[
            jax.ShapeDtypeStruct((B, 1, _FRAME_TOPK), jnp.int32),
            jax.ShapeDtypeStruct((B, 1, T), jnp.float32),
        ],
    )(frame_repr)
    frame_idx = frame_idx3.reshape(B, _FRAME_TOPK)
    frame_mask = frame_mask3.reshape(B, T)
    fidx_flat = frame_idx.reshape(-1)

    d2, d0, f_all = pl.pallas_call(
        _d2_kernel,
        grid_spec=pltpu.PrefetchScalarGridSpec(
            num_scalar_prefetch=1,
            grid=(P,),
            in_specs=[
                pl.BlockSpec(
                    (1, 1, N, D),
                    lambda p, idx_ref: (p // _FRAME_TOPK, idx_ref[p], 0, 0)),
            ],
            out_specs=[
                pl.BlockSpec((1, _NP, _NP), lambda p, idx_ref: (p, 0, 0)),
                pl.BlockSpec((1, 1, _NP), lambda p, idx_ref: (p, 0, 0)),
                pl.BlockSpec((1, _NP, D), lambda p, idx_ref: (p, 0, 0)),
            ],
        ),
        out_shape=[
            jax.ShapeDtypeStruct((P, _NP, _NP), jnp.float32),
            jax.ShapeDtypeStruct((P, 1, _NP), jnp.float32),
            jax.ShapeDtypeStruct((P, _NP, D), jnp.float32),
        ],
    )(fidx_flat, x)

    tidx, cm, zflat = _run_fps_tok_sc(d2, d0, f_all.reshape(P * _NP, D))
    token_idx = tidx.reshape(B, _FRAME_TOPK, 64)[:, :, :K]
    z = zflat.reshape(B, _FRAME_TOPK, 64, D)[:, :, :K]

    token_mask = pl.pallas_call(
        _mask_place_kernel,
        grid=(B,),
        in_specs=[
            pl.BlockSpec((1, 1, _FRAME_TOPK), lambda b: (b, 0, 0)),
            pl.BlockSpec((1, _FRAME_TOPK, N), lambda b: (b, 0, 0)),
        ],
        out_specs=pl.BlockSpec((1, T, N), lambda b: (b, 0, 0)),
        out_shape=jax.ShapeDtypeStruct((B, T, N), jnp.float32),
    )(frame_idx3, cm.reshape(B, _FRAME_TOPK, _NP)[:, :, :N])

    return z, frame_idx, token_idx, frame_mask, token_mask


# speedup vs baseline: 5.1032x; 5.1032x over previous
import jax
import jax.numpy as jnp
from jax import lax
from jax.experimental import pallas as pl
from jax.experimental.pallas import tpu as pltpu
from jax.experimental.pallas import tpu_sc as plsc

_FRAME_TOPK = 16
_TOKEN_TOPK = 49
_EMA_ALPHA = 0.9
_NP = 208
_NEG = -3.0e38


def _argmax_col(v, iota_col, size):
    m = jnp.max(v, axis=0, keepdims=True)
    cand = jnp.where(v == m, iota_col, size)
    idx = jnp.min(cand, axis=0, keepdims=True)
    oh = (iota_col == idx).astype(jnp.float32)
    return oh, idx


def _dot(a, b, precision=jax.lax.Precision.HIGHEST):
    return jax.lax.dot_general(a, b, (((1,), (0,)), ((), ())),
                               preferred_element_type=jnp.float32,
                               precision=precision)


def _frame_repr_kernel(x_ref, out_ref):
    out_ref[...] = jnp.sum(x_ref[...], axis=1, keepdims=True) / 196.0


def _frame_sel_kernel(fr_ref, idx_ref, mask_ref):
    T = 64
    fr = fr_ref[0]
    iota_col = jax.lax.broadcasted_iota(jnp.int32, (T, 1), 0)
    iota_row = jax.lax.broadcasted_iota(jnp.int32, (1, T), 1)
    iota_k = jax.lax.broadcasted_iota(jnp.int32, (1, _FRAME_TOPK), 1)

    ema = fr_ref[0, 0:1, :]
    nov = jnp.zeros((T, 1), dtype=jnp.float32)
    for t in range(T):
        ft = fr_ref[0, t:t + 1, :]
        d = ft - ema
        n = jnp.sum(d * d)
        nov = jnp.where(iota_col == t, n, nov)
        ema = _EMA_ALPHA * ema + (1.0 - _EMA_ALPHA) * ft

    oh, idx = _argmax_col(nov, iota_col, T)
    idx_acc = jnp.where(iota_k == 0, idx, 0)
    mask_row = (iota_row == idx).astype(jnp.float32)
    min_d = jnp.full((T, 1), jnp.inf, dtype=jnp.float32)

    for i in range(1, _FRAME_TOPK):
        last = jnp.sum(oh * fr, axis=0, keepdims=True)
        diff = fr - last
        dist = jnp.sum(diff * diff, axis=1, keepdims=True)
        min_d = jnp.minimum(min_d, dist)
        min_d = jnp.where(oh > 0, -1.0, min_d)
        oh, idx = _argmax_col(min_d, iota_col, T)
        idx_acc = jnp.where(iota_k == i, idx, idx_acc)
        mask_row = mask_row + (iota_row == idx).astype(jnp.float32)
    idx_ref[0] = idx_acc
    mask_ref[0] = mask_row


def _d2_kernel(f_ref, d2_ref, d0_ref):
    Fin = f_ref[0]
    ri = jax.lax.broadcasted_iota(jnp.int32, (_NP, 1), 0)
    Fp = jnp.where(ri < 196, Fin, 0.0)
    g = jax.lax.dot_general(Fp, Fp, (((1,), (1,)), ((), ())),
                            preferred_element_type=jnp.float32,
                            precision=jax.lax.Precision.HIGHEST)
    ii = jax.lax.broadcasted_iota(jnp.int32, (_NP, _NP), 0)
    jj = jax.lax.broadcasted_iota(jnp.int32, (_NP, _NP), 1)
    eye = (ii == jj).astype(jnp.float32)
    rn_col = jnp.sum(g * eye, axis=1, keepdims=True)
    rn_row = jnp.sum(g * eye, axis=0, keepdims=True)
    d2 = rn_col + rn_row - 2.0 * g
    pad = jnp.logical_or(ii >= 196, jj >= 196)
    d2_ref[0] = jnp.where(pad, _NEG, d2)

    gm_row = jnp.sum(g, axis=0, keepdims=True) * (1.0 / 196.0)
    musq = jnp.sum(g) * (1.0 / (196.0 * 196.0))
    d0 = rn_row - 2.0 * gm_row + musq
    jr = jax.lax.broadcasted_iota(jnp.int32, (1, _NP), 1)
    d0_ref[0] = jnp.where(jr >= 196, _NEG, d0)


def _mask_place_kernel(fidx_ref, cm_ref, out_ref):
    T = 64
    fidx = fidx_ref[0]
    cm = cm_ref[0]
    iota_col = jax.lax.broadcasted_iota(jnp.int32, (T, _FRAME_TOPK), 0)
    p = (iota_col == fidx).astype(jnp.float32)
    out_ref[0] = _dot(p, cm, precision=jax.lax.Precision.DEFAULT)


def _frame_gather_sc_body(off_hbm, x_hbm, f_out_hbm, off_v, idx_v, f_v, sem):
    C = _NP // 16
    p = lax.axis_index("s") * 2 + lax.axis_index("c")
    pltpu.sync_copy(off_hbm.at[p], off_v)
    base = off_v[...][0]
    iota = lax.iota(jnp.int32, 16)
    for g in range(C):
        n = jnp.minimum(iota + (g * 16), 195)
        idx_v[pl.ds(g * 16, 16)] = base + n * 64
    half = _NP // 2
    for h in range(2):
        pltpu.async_copy(
            x_hbm.at[idx_v.at[pl.ds(h * half, half)]], f_v, sem).wait()
        pltpu.sync_copy(f_v, f_out_hbm.at[p, pl.ds(h * half, half)])


def _run_frame_gather_sc(offs, x_rows):
    D = x_rows.shape[1]
    P = offs.shape[0]
    mesh = plsc.VectorSubcoreMesh(core_axis_name="c", subcore_axis_name="s")
    gather = pl.kernel(
        _frame_gather_sc_body,
        out_type=jax.ShapeDtypeStruct((P, _NP, D), jnp.float32),
        mesh=mesh,
        scratch_types=[
            pltpu.VMEM((16,), jnp.int32),
            pltpu.VMEM((_NP,), jnp.int32),
            pltpu.VMEM((_NP // 2, D), jnp.float32),
            pltpu.SemaphoreType.DMA,
        ],
    )
    return gather(offs, x_rows)


def _fps_tok_sc_body(d2_hbm, d0_hbm, f_hbm,
                     tidx_hbm, cm_hbm, z_hbm,
                     d2_v, d0_v, tidx_v, idx_v, cm_v, z_v, sem):
    K = _TOKEN_TOPK
    C = _NP // 16
    p = lax.axis_index("s") * 2 + lax.axis_index("c")
    pltpu.sync_copy(d2_hbm.at[p], d2_v)
    pltpu.sync_copy(d0_hbm.at[p], d0_v)
    iota = lax.iota(jnp.int32, 16)
    off = p * _NP

    def store_tidx(k, sel):
        for c in range(4):
            cur = tidx_v[pl.ds(c * 16, 16)]
            tidx_v[pl.ds(c * 16, 16)] = jnp.where(
                iota + (c * 16) == k, sel, cur)

    for c in range(4):
        tidx_v[pl.ds(c * 16, 16)] = jnp.zeros((16,), jnp.int32)

    def argmax_chunks(vecs):
        vmax = vecs[0]
        for v in vecs[1:]:
            vmax = jnp.maximum(vmax, v)
        gmax = vmax[0]
        for l in range(1, 16):
            gmax = jnp.maximum(gmax, vmax[l])
        cmin = jnp.where(vecs[0] == gmax, iota, _NP)
        for c in range(1, len(vecs)):
            cand = jnp.where(vecs[c] == gmax, iota + (c * 16), _NP)
            cmin = jnp.minimum(cmin, cand)
        sel = cmin[0]
        for l in range(1, 16):
            sel = jnp.minimum(sel, cmin[l])
        return sel

    sel0 = argmax_chunks([d0_v[pl.ds(c * 16, 16)] for c in range(C)])
    store_tidx(0, sel0)
    inf16 = jnp.full((16,), jnp.inf, jnp.float32)

    def body(k, carry):
        sel = carry[0]
        mins = carry[1:]
        new_mins = []
        for c in range(C):
            row = d2_v[sel, pl.ds(c * 16, 16)]
            m = jnp.minimum(mins[c], row)
            m = jnp.where(iota + (c * 16) == sel, -1.0, m)
            new_mins.append(m)
        new = argmax_chunks(new_mins)
        store_tidx(k, new)
        return (new, *new_mins)

    final = lax.fori_loop(1, K, body, (sel0, *([inf16] * C)))
    last_sel = final[0]
    for c in range(C):
        m = jnp.where(iota + (c * 16) == last_sel, -1.0, final[1 + c])
        cm_v[pl.ds(c * 16, 16)] = jnp.where(m == -1.0, 1.0, 0.0)

    for c in range(4):
        idx_v[pl.ds(c * 16, 16)] = tidx_v[pl.ds(c * 16, 16)] + off
    pltpu.async_copy(f_hbm.at[idx_v], z_v, sem).wait()

    pltpu.sync_copy(z_v, z_hbm.at[p])
    pltpu.sync_copy(cm_v, cm_hbm.at[p])
    pltpu.sync_copy(tidx_v, tidx_hbm.at[p])


def _run_fps_tok_sc(d2, d0, f_all):
    P = d2.shape[0]
    D = f_all.shape[1]
    mesh = plsc.VectorSubcoreMesh(core_axis_name="c", subcore_axis_name="s")
    fps = pl.kernel(
        _fps_tok_sc_body,
        out_type=[
            jax.ShapeDtypeStruct((P, 64), jnp.int32),
            jax.ShapeDtypeStruct((P, _NP), jnp.float32),
            jax.ShapeDtypeStruct((P, 64, D), jnp.float32),
        ],
        mesh=mesh,
        scratch_types=[
            pltpu.VMEM((_NP, _NP), jnp.float32),
            pltpu.VMEM((_NP,), jnp.float32),
            pltpu.VMEM((64,), jnp.int32),
            pltpu.VMEM((64,), jnp.int32),
            pltpu.VMEM((_NP,), jnp.float32),
            pltpu.VMEM((64, D), jnp.float32),
            pltpu.SemaphoreType.DMA,
        ],
    )
    return fps(d2, d0.reshape(P, _NP), f_all)


def kernel(x):
    B, T, N, D = x.shape
    K = _TOKEN_TOPK
    P = B * _FRAME_TOPK

    xt = jnp.transpose(x, (0, 2, 1, 3))
    x_rows = xt.reshape(B * N * T, D)

    frame_repr = pl.pallas_call(
        _frame_repr_kernel,
        grid=(B, T // 8),
        in_specs=[pl.BlockSpec((1, N, 8, D), lambda b, t: (b, 0, t, 0))],
        out_specs=pl.BlockSpec((1, 1, 8, D), lambda b, t: (b, 0, t, 0)),
        out_shape=jax.ShapeDtypeStruct((B, 1, T, D), jnp.float32),
    )(xt).reshape(B, T, D)

    frame_idx3, frame_mask3 = pl.pallas_call(
        _frame_sel_kernel,
        grid=(B,),
        in_specs=[pl.BlockSpec((1, T, D), lambda b: (b, 0, 0))],
        out_specs=[
            pl.BlockSpec((1, 1, _FRAME_TOPK), lambda b: (b, 0, 0)),
            pl.BlockSpec((1, 1, T), lambda b: (b, 0, 0)),
        ],
        out_shape=[
            jax.ShapeDtypeStruct((B, 1, _FRAME_TOPK), jnp.int32),
            jax.ShapeDtypeStruct((B, 1, T), jnp.float32),
        ],
    )(frame_repr)
    frame_idx = frame_idx3.reshape(B, _FRAME_TOPK)
    frame_mask = frame_mask3.reshape(B, T)

    offs = jnp.arange(B, dtype=jnp.int32)[:, None] * (N * T) + frame_idx
    offs = jnp.broadcast_to(offs.reshape(P)[:, None], (P, 16))
    f_all = _run_frame_gather_sc(offs, x_rows)

    d2, d0 = pl.pallas_call(
        _d2_kernel,
        grid=(P,),
        in_specs=[pl.BlockSpec((1, _NP, D), lambda p: (p, 0, 0))],
        out_specs=[
            pl.BlockSpec((1, _NP, _NP), lambda p: (p, 0, 0)),
            pl.BlockSpec((1, 1, _NP), lambda p: (p, 0, 0)),
        ],
        out_shape=[
            jax.ShapeDtypeStruct((P, _NP, _NP), jnp.float32),
            jax.ShapeDtypeStruct((P, 1, _NP), jnp.float32),
        ],
    )(f_all)

    tidx, cm, zflat = _run_fps_tok_sc(d2, d0, f_all.reshape(P * _NP, D))
    token_idx = tidx.reshape(B, _FRAME_TOPK, 64)[:, :, :K]
    z = zflat.reshape(B, _FRAME_TOPK, 64, D)[:, :, :K]

    token_mask = pl.pallas_call(
        _mask_place_kernel,
        grid=(B,),
        in_specs=[
            pl.BlockSpec((1, 1, _FRAME_TOPK), lambda b: (b, 0, 0)),
            pl.BlockSpec((1, _FRAME_TOPK, N), lambda b: (b, 0, 0)),
        ],
        out_specs=pl.BlockSpec((1, T, N), lambda b: (b, 0, 0)),
        out_shape=jax.ShapeDtypeStruct((B, T, N), jnp.float32),
    )(frame_idx3, cm.reshape(B, _FRAME_TOPK, _NP)[:, :, :N])

    return z, frame_idx, token_idx, frame_mask, token_mask
